# Initial kernel scaffold; baseline (speedup 1.0000x reference)
import jax
import jax.numpy as jnp
from jax.experimental import pallas as pl


def kernel(state, W, b, item_embeds):
    raise NotImplementedError("write your pallas kernel here")


# speedup vs baseline: 1.9493x; 1.9493x over previous
import functools

import jax
import jax.numpy as jnp
from jax.experimental import pallas as pl
from jax.experimental.pallas import tpu as pltpu

B_TILE = 256
V_TILE = 2048
TOPK = 10
RUN_W = 128


def _body(state_ref, w_ref, b_ref, ie_ref, out_ref, rv_ref, ri_ref,
          *, vocab, v_tiles):
    j = pl.program_id(1)

    @pl.when(j == 0)
    def _init():
        rv_ref[...] = jnp.full_like(rv_ref[...], -jnp.inf)
        ri_ref[...] = jnp.zeros_like(ri_ref[...])

    x = state_ref[...]
    act = jnp.tanh(jnp.dot(x, w_ref[...], preferred_element_type=jnp.float32)
                   + b_ref[...])
    act = act / jnp.sqrt(jnp.sum(act * act, axis=1, keepdims=True))

    ie = ie_ref[...]
    ien = ie / jnp.sqrt(jnp.sum(ie * ie, axis=0, keepdims=True))

    s = jnp.dot(act, ien, preferred_element_type=jnp.float32)

    col = jax.lax.broadcasted_iota(jnp.int32, (B_TILE, V_TILE), 1) + j * V_TILE
    s = jnp.where(col < vocab, s, -jnp.inf)

    cv = jnp.concatenate([s, rv_ref[...]], axis=1)
    ci = jnp.concatenate([col, ri_ref[...]], axis=1)

    vs, ks = [], []
    for _ in range(TOPK):
        m = jnp.max(cv, axis=1, keepdims=True)
        pick = jnp.min(jnp.where(cv == m, ci, jnp.int32(2**31 - 1)),
                       axis=1, keepdims=True)
        vs.append(m)
        ks.append(pick)
        cv = jnp.where(ci == pick, -jnp.inf, cv)

    nv = jnp.concatenate(vs, axis=1)
    ni = jnp.concatenate(ks, axis=1)
    rv_ref[...] = jnp.concatenate(
        [nv, jnp.full((nv.shape[0], RUN_W - TOPK), -jnp.inf, jnp.float32)],
        axis=1)
    ri_ref[...] = jnp.concatenate(
        [ni, jnp.zeros((ni.shape[0], RUN_W - TOPK), jnp.int32)], axis=1)

    @pl.when(j == v_tiles - 1)
    def _emit():
        out_ref[...] = ni


def kernel(state, W, b, item_embeds):
    batch, state_dim = state.shape
    vocab, dim = item_embeds.shape
    ie_t = item_embeds.T
    b2 = b.reshape(1, dim)

    n_b = batch // B_TILE
    n_v = pl.cdiv(vocab, V_TILE)

    out = pl.pallas_call(
        functools.partial(_body, vocab=vocab, v_tiles=n_v),
        grid=(n_b, n_v),
        in_specs=[
            pl.BlockSpec((B_TILE, state_dim), lambda i, j: (i, 0)),
            pl.BlockSpec((state_dim, dim), lambda i, j: (0, 0)),
            pl.BlockSpec((1, dim), lambda i, j: (0, 0)),
            pl.BlockSpec((dim, V_TILE), lambda i, j: (0, j)),
        ],
        out_specs=pl.BlockSpec((B_TILE, TOPK), lambda i, j: (i, 0)),
        out_shape=jax.ShapeDtypeStruct((batch, TOPK), jnp.int32),
        scratch_shapes=[
            pltpu.VMEM((B_TILE, RUN_W), jnp.float32),
            pltpu.VMEM((B_TILE, RUN_W), jnp.int32),
        ],
    )(state, W, b2, ie_t)
    return out


# speedup vs baseline: 2.1881x; 1.1225x over previous
import functools

import jax
import jax.numpy as jnp
from jax.experimental import pallas as pl
from jax.experimental.pallas import tpu as pltpu

B_TILE = 256
V_TILE = 2048
TOPK = 10
RUN_W = 128
IMAX = 2**31 - 1


def _body(state_ref, w_ref, b_ref, ie_ref, out_ref,
          s_scr, cv_scr, ci_scr, rv_ref, ri_ref, *, vocab, v_tiles):
    j = pl.program_id(1)

    @pl.when(j == 0)
    def _init():
        rv_ref[...] = jnp.full_like(rv_ref[...], -jnp.inf)
        ri_ref[...] = jnp.zeros_like(ri_ref[...])

    x = state_ref[...]
    act = jnp.tanh(jnp.dot(x, w_ref[...], preferred_element_type=jnp.float32)
                   + b_ref[...])
    act = act / jnp.sqrt(jnp.sum(act * act, axis=1, keepdims=True))

    ie = ie_ref[...]
    ien = ie / jnp.sqrt(jnp.sum(ie * ie, axis=0, keepdims=True))

    s = jnp.dot(act, ien, preferred_element_type=jnp.float32)
    col = jax.lax.broadcasted_iota(jnp.int32, s.shape, 1) + j * V_TILE
    s = jnp.where(col < vocab, s, -jnp.inf)
    s_scr[...] = s

    v10 = rv_ref[:, TOPK - 1:TOPK]
    nhit = jnp.sum((s > v10).astype(jnp.int32), axis=1, keepdims=True)
    npass = jnp.minimum(jnp.max(nhit), TOPK)

    cv_scr[...] = jnp.full_like(cv_scr[...], -jnp.inf)
    ci_scr[...] = jnp.zeros_like(ci_scr[...])
    lane = jax.lax.broadcasted_iota(jnp.int32, (B_TILE, RUN_W), 1)

    for t in range(TOPK):
        @pl.when(t < npass)
        def _extract(t=t):
            sv = s_scr[...]
            m = jnp.max(sv, axis=1, keepdims=True)
            pick = jnp.min(jnp.where(sv == m, col, IMAX),
                           axis=1, keepdims=True)
            s_scr[...] = jnp.where(col == pick, -jnp.inf, sv)
            cv_scr[...] = jnp.where(lane == t, m, cv_scr[...])
            ci_scr[...] = jnp.where(lane == t, pick, ci_scr[...])

    @pl.when(npass > 0)
    def _merge():
        mv = jnp.concatenate([cv_scr[...], rv_ref[...]], axis=1)
        mi = jnp.concatenate([ci_scr[...], ri_ref[...]], axis=1)
        vs, ks = [], []
        for _ in range(TOPK):
            m = jnp.max(mv, axis=1, keepdims=True)
            pick = jnp.min(jnp.where(mv == m, mi, IMAX),
                           axis=1, keepdims=True)
            vs.append(m)
            ks.append(pick)
            mv = jnp.where(mi == pick, -jnp.inf, mv)
        nv = jnp.concatenate(vs, axis=1)
        ni = jnp.concatenate(ks, axis=1)
        rv_ref[...] = jnp.concatenate(
            [nv, jnp.full((B_TILE, RUN_W - TOPK), -jnp.inf, jnp.float32)],
            axis=1)
        ri_ref[...] = jnp.concatenate(
            [ni, jnp.zeros((B_TILE, RUN_W - TOPK), jnp.int32)], axis=1)

    @pl.when(j == v_tiles - 1)
    def _emit():
        out_ref[...] = ri_ref[:, :TOPK]


def kernel(state, W, b, item_embeds):
    batch, state_dim = state.shape
    vocab, dim = item_embeds.shape
    ie_t = item_embeds.T
    b2 = b.reshape(1, dim)

    n_b = batch // B_TILE
    n_v = pl.cdiv(vocab, V_TILE)

    out = pl.pallas_call(
        functools.partial(_body, vocab=vocab, v_tiles=n_v),
        grid=(n_b, n_v),
        in_specs=[
            pl.BlockSpec((B_TILE, state_dim), lambda i, j: (i, 0)),
            pl.BlockSpec((state_dim, dim), lambda i, j: (0, 0)),
            pl.BlockSpec((1, dim), lambda i, j: (0, 0)),
            pl.BlockSpec((dim, V_TILE), lambda i, j: (0, j)),
        ],
        out_specs=pl.BlockSpec((B_TILE, TOPK), lambda i, j: (i, 0)),
        out_shape=jax.ShapeDtypeStruct((batch, TOPK), jnp.int32),
        scratch_shapes=[
            pltpu.VMEM((B_TILE, V_TILE), jnp.float32),
            pltpu.VMEM((B_TILE, RUN_W), jnp.float32),
            pltpu.VMEM((B_TILE, RUN_W), jnp.int32),
            pltpu.VMEM((B_TILE, RUN_W), jnp.float32),
            pltpu.VMEM((B_TILE, RUN_W), jnp.int32),
        ],
    )(state, W, b2, ie_t)
    return out


# speedup vs baseline: 2.9648x; 1.3550x over previous
import functools

import jax
import jax.numpy as jnp
from jax.experimental import pallas as pl
from jax.experimental.pallas import tpu as pltpu

B_TILE = 256
V_TILE = 4096
SUB_W = 1024
N_SUB = V_TILE // SUB_W
TOPK = 10
RUN_W = 128
MERGE_AT = RUN_W - 2 * TOPK
IMAX = 2**31 - 1


def _body(state_ref, w_ref, b_ref, ie_ref, out_ref,
          s_scr, cv_scr, ci_scr, rv_ref, ri_ref, base_ref,
          *, vocab, v_tiles):
    j = pl.program_id(1)

    @pl.when(j == 0)
    def _init():
        rv_ref[...] = jnp.full_like(rv_ref[...], -jnp.inf)
        ri_ref[...] = jnp.zeros_like(ri_ref[...])
        cv_scr[...] = jnp.full_like(cv_scr[...], -jnp.inf)
        ci_scr[...] = jnp.zeros_like(ci_scr[...])
        base_ref[0] = 0

    x = state_ref[...]
    act = jnp.tanh(jnp.dot(x, w_ref[...], preferred_element_type=jnp.float32)
                   + b_ref[...])
    act = act / jnp.sqrt(jnp.sum(act * act, axis=1, keepdims=True))

    ie = ie_ref[...]
    ien = ie / jnp.sqrt(jnp.sum(ie * ie, axis=0, keepdims=True))

    s = jnp.dot(act, ien, preferred_element_type=jnp.float32)
    col = jax.lax.broadcasted_iota(jnp.int32, s.shape, 1) + j * V_TILE
    s = jnp.where(col < vocab, s, -jnp.inf)
    s_scr[...] = s

    lane = jax.lax.broadcasted_iota(jnp.int32, (B_TILE, RUN_W), 1)

    def merge():
        mv = jnp.concatenate([cv_scr[...], rv_ref[...]], axis=1)
        mi = jnp.concatenate([ci_scr[...], ri_ref[...]], axis=1)
        vs, ks = [], []
        for _ in range(TOPK):
            m = jnp.max(mv, axis=1, keepdims=True)
            pick = jnp.min(jnp.where(mv == m, mi, IMAX),
                           axis=1, keepdims=True)
            vs.append(m)
            ks.append(pick)
            mv = jnp.where(mi == pick, -jnp.inf, mv)
        nv = jnp.concatenate(vs, axis=1)
        ni = jnp.concatenate(ks, axis=1)
        rv_ref[...] = jnp.concatenate(
            [nv, jnp.full((B_TILE, RUN_W - TOPK), -jnp.inf, jnp.float32)],
            axis=1)
        ri_ref[...] = jnp.concatenate(
            [ni, jnp.zeros((B_TILE, RUN_W - TOPK), jnp.int32)], axis=1)
        cv_scr[...] = jnp.full_like(cv_scr[...], -jnp.inf)
        ci_scr[...] = jnp.zeros_like(ci_scr[...])
        base_ref[0] = 0

    for k in range(N_SUB):
        sub = s[:, k * SUB_W:(k + 1) * SUB_W]
        subcol = col[:, k * SUB_W:(k + 1) * SUB_W]
        v10 = rv_ref[:, TOPK - 1:TOPK]
        nhit = jnp.sum((sub > v10).astype(jnp.int32), axis=1, keepdims=True)
        npass = jnp.minimum(jnp.max(nhit), TOPK)
        base = base_ref[0]

        for t in range(TOPK):
            @pl.when(t < npass)
            def _extract(t=t, k=k):
                sv = s_scr[:, k * SUB_W:(k + 1) * SUB_W]
                sc = col[:, k * SUB_W:(k + 1) * SUB_W]
                m = jnp.max(sv, axis=1, keepdims=True)
                pick = jnp.min(jnp.where(sv == m, sc, IMAX),
                               axis=1, keepdims=True)
                s_scr[:, k * SUB_W:(k + 1) * SUB_W] = (
                    jnp.where(sc == pick, -jnp.inf, sv))
                cv_scr[...] = jnp.where(lane == base + t, m, cv_scr[...])
                ci_scr[...] = jnp.where(lane == base + t, pick, ci_scr[...])

        base_ref[0] = base + npass

        last = (j == v_tiles - 1) if k == N_SUB - 1 else False
        do_merge = (base_ref[0] > MERGE_AT) | last if k == N_SUB - 1 else \
            (base_ref[0] > MERGE_AT)

        @pl.when(do_merge)
        def _do_merge():
            merge()

    @pl.when(j == v_tiles - 1)
    def _emit():
        out_ref[...] = ri_ref[:, :TOPK]


def kernel(state, W, b, item_embeds):
    batch, state_dim = state.shape
    vocab, dim = item_embeds.shape
    ie_t = item_embeds.T
    b2 = b.reshape(1, dim)

    n_b = batch // B_TILE
    n_v = pl.cdiv(vocab, V_TILE)

    out = pl.pallas_call(
        functools.partial(_body, vocab=vocab, v_tiles=n_v),
        grid=(n_b, n_v),
        in_specs=[
            pl.BlockSpec((B_TILE, state_dim), lambda i, j: (i, 0)),
            pl.BlockSpec((state_dim, dim), lambda i, j: (0, 0)),
            pl.BlockSpec((1, dim), lambda i, j: (0, 0)),
            pl.BlockSpec((dim, V_TILE), lambda i, j: (0, j)),
        ],
        out_specs=pl.BlockSpec((B_TILE, TOPK), lambda i, j: (i, 0)),
        out_shape=jax.ShapeDtypeStruct((batch, TOPK), jnp.int32),
        scratch_shapes=[
            pltpu.VMEM((B_TILE, V_TILE), jnp.float32),
            pltpu.VMEM((B_TILE, RUN_W), jnp.float32),
            pltpu.VMEM((B_TILE, RUN_W), jnp.int32),
            pltpu.VMEM((B_TILE, RUN_W), jnp.float32),
            pltpu.VMEM((B_TILE, RUN_W), jnp.int32),
            pltpu.SMEM((1,), jnp.int32),
        ],
    )(state, W, b2, ie_t)
    return out
